# Initial kernel scaffold; baseline (speedup 1.0000x reference)
import jax
import jax.numpy as jnp
from jax.experimental import pallas as pl


def kernel(x, edge_index, W_l1, W_r1, b1, W_l2, W_r2, b2, W_l3, W_r3, b3, W_fc, b_fc):
    raise NotImplementedError("write your pallas kernel here")


# speedup vs baseline: 4.7624x; 4.7624x over previous
import functools

import jax
import jax.numpy as jnp
from jax import lax
from jax.experimental import pallas as pl
from jax.experimental.pallas import tpu as pltpu
from jax.experimental.pallas import tpu_sc as plsc

_N = 10000
_D = 128
_E = 320000
_DOUT = 64

_NC = 2
_NS = 16
_NW = _NC * _NS
_EPW = _E // _NW
_CHUNK = 80
_STEPS = _EPW // _CHUNK
_NP = 10240
_RPT = _NP // _NS


def _make_agg(with_gather):
  mesh = plsc.VectorSubcoreMesh(core_axis_name="c", subcore_axis_name="s")
  out_type = jax.ShapeDtypeStruct((_NC * _NP, _D), jnp.float32)
  scratch = [
      pltpu.VMEM_SHARED((_NP, _D), jnp.float32),
      pltpu.VMEM((_CHUNK,), jnp.int32),
      pltpu.VMEM((_CHUNK, _D), jnp.float32),
      pltpu.SemaphoreType.DMA,
  ]
  if with_gather:
    scratch.append(pltpu.VMEM((_CHUNK,), jnp.int32))

  def body(*refs):
    if with_gather:
      (table, srcs, dsts, z128, o_acc,
       acc_s, dst_v, rows_v, sem, src_v) = refs
    else:
      (ones, dsts, z128, o_acc,
       acc_s, dst_v, rows_v, sem) = refs
    c = lax.axis_index("c")
    s = lax.axis_index("s")
    wid = s * _NC + c
    r0 = s * _RPT

    pltpu.sync_copy(z128, rows_v)
    for k in range(_RPT // _CHUNK):
      pltpu.sync_copy(rows_v, acc_s.at[pl.ds(r0 + k * _CHUNK, _CHUNK)])
    if not with_gather:
      pltpu.sync_copy(ones, rows_v)
    plsc.subcore_barrier()

    e0 = wid * _EPW

    def step(j, carry):
      base = e0 + j * _CHUNK
      pltpu.sync_copy(dsts.at[pl.ds(base, _CHUNK)], dst_v)
      if with_gather:
        pltpu.sync_copy(srcs.at[pl.ds(base, _CHUNK)], src_v)
        pltpu.async_copy(table.at[src_v], rows_v, sem).wait()
      pltpu.sync_copy(rows_v, acc_s.at[dst_v], add=True)
      return carry
    lax.fori_loop(0, _STEPS, step, 0)
    plsc.subcore_barrier()

    for k in range(_RPT // _CHUNK):
      pltpu.sync_copy(acc_s.at[pl.ds(r0 + k * _CHUNK, _CHUNK)], rows_v)
      pltpu.sync_copy(rows_v, o_acc.at[pl.ds(c * _NP + r0 + k * _CHUNK, _CHUNK)])

  return pl.kernel(body, out_type=out_type, mesh=mesh, scratch_types=scratch)


_agg = _make_agg(True)
_deg_pass = _make_agg(False)

_RB = 2000


def _conv_body(acc_ref, deg_ref, x_ref, wl_ref, wr_ref, b_ref, o_ref, *, relu):
  ssum = acc_ref[0] + acc_ref[1]
  deg = deg_ref[0, :, 0:1] + deg_ref[1, :, 0:1]
  mean = ssum / jnp.maximum(deg, 1.0)
  out = (jnp.dot(mean, wl_ref[...], preferred_element_type=jnp.float32)
         + jnp.dot(x_ref[...], wr_ref[...], preferred_element_type=jnp.float32)
         + b_ref[...])
  nrm = jnp.sqrt(jnp.sum(out * out, axis=-1, keepdims=True))
  out = out / jnp.maximum(nrm, 1e-12)
  if relu:
    out = jnp.maximum(out, 0.0)
  o_ref[...] = out


def _final_body(acc_ref, deg_ref, x_ref, wl_ref, wr_ref, b_ref, wfc_ref,
                bfc_ref, o_ref):
  ssum = acc_ref[0] + acc_ref[1]
  deg = deg_ref[0, :, 0:1] + deg_ref[1, :, 0:1]
  mean = ssum / jnp.maximum(deg, 1.0)
  out = (jnp.dot(mean, wl_ref[...], preferred_element_type=jnp.float32)
         + jnp.dot(x_ref[...], wr_ref[...], preferred_element_type=jnp.float32)
         + b_ref[...])
  nrm = jnp.sqrt(jnp.sum(out * out, axis=-1, keepdims=True))
  out = out / jnp.maximum(nrm, 1e-12)
  logits = (jnp.dot(out, wfc_ref[...], preferred_element_type=jnp.float32)
            + bfc_ref[...])
  m = jnp.max(logits, axis=-1, keepdims=True)
  e = jnp.exp(logits - m)
  o_ref[...] = e / jnp.sum(e, axis=-1, keepdims=True)


def _conv(acc, deg, x, wl, wr, b, relu):
  grid = (_N // _RB,)
  return pl.pallas_call(
      functools.partial(_conv_body, relu=relu),
      grid=grid,
      in_specs=[
          pl.BlockSpec((_NC, _RB, _D), lambda i: (0, i, 0)),
          pl.BlockSpec((_NC, _RB, _D), lambda i: (0, i, 0)),
          pl.BlockSpec((_RB, _D), lambda i: (i, 0)),
          pl.BlockSpec((_D, _D), lambda i: (0, 0)),
          pl.BlockSpec((_D, _D), lambda i: (0, 0)),
          pl.BlockSpec((1, _D), lambda i: (0, 0)),
      ],
      out_specs=pl.BlockSpec((_RB, _D), lambda i: (i, 0)),
      out_shape=jax.ShapeDtypeStruct((_N, _D), jnp.float32),
  )(acc, deg, x, wl, wr, b)


def _final(acc, deg, x, wl, wr, b, wfc, bfc):
  grid = (_N // _RB,)
  return pl.pallas_call(
      _final_body,
      grid=grid,
      in_specs=[
          pl.BlockSpec((_NC, _RB, _D), lambda i: (0, i, 0)),
          pl.BlockSpec((_NC, _RB, _D), lambda i: (0, i, 0)),
          pl.BlockSpec((_RB, _D), lambda i: (i, 0)),
          pl.BlockSpec((_D, _D), lambda i: (0, 0)),
          pl.BlockSpec((_D, _D), lambda i: (0, 0)),
          pl.BlockSpec((1, _D), lambda i: (0, 0)),
          pl.BlockSpec((_D, _DOUT), lambda i: (0, 0)),
          pl.BlockSpec((1, _DOUT), lambda i: (0, 0)),
      ],
      out_specs=pl.BlockSpec((_RB, _DOUT), lambda i: (i, 0)),
      out_shape=jax.ShapeDtypeStruct((_N, _DOUT), jnp.float32),
  )(acc, deg, x, wl, wr, b, wfc, bfc)


def kernel(x, edge_index, W_l1, W_r1, b1, W_l2, W_r2, b2, W_l3, W_r3, b3,
           W_fc, b_fc):
  src = edge_index[0].astype(jnp.int32)
  dst = edge_index[1].astype(jnp.int32)
  z128 = jnp.zeros((_CHUNK, _D), jnp.float32)
  ones = jnp.ones((_CHUNK, _D), jnp.float32)

  deg = _deg_pass(ones, dst, z128).reshape(_NC, _NP, _D)
  acc1 = _agg(x, src, dst, z128).reshape(_NC, _NP, _D)
  h1 = _conv(acc1, deg, x, W_l1, W_r1, b1.reshape(1, _D), relu=True)
  acc2 = _agg(h1, src, dst, z128).reshape(_NC, _NP, _D)
  h2 = _conv(acc2, deg, h1, W_l2, W_r2, b2.reshape(1, _D), relu=True)
  acc3 = _agg(h2, src, dst, z128).reshape(_NC, _NP, _D)
  return _final(acc3, deg, h2, W_l3, W_r3, b3.reshape(1, _D),
                W_fc, b_fc.reshape(1, _DOUT))


# speedup vs baseline: 6.2322x; 1.3086x over previous
import functools

import jax
import jax.numpy as jnp
from jax import lax
from jax.experimental import pallas as pl
from jax.experimental.pallas import tpu as pltpu
from jax.experimental.pallas import tpu_sc as plsc

_N = 10000
_D = 128
_E = 320000
_DOUT = 64

_NC = 2
_NS = 16
_NW = _NC * _NS
_EPW = _E // _NW
_CHUNK = 80
_STEPS = _EPW // _CHUNK
_NP = 10240
_RPT = _NP // _NS


def _make_agg(with_gather):
  mesh = plsc.VectorSubcoreMesh(core_axis_name="c", subcore_axis_name="s")
  out_type = jax.ShapeDtypeStruct((_NC * _NP, _D), jnp.float32)
  scratch = [
      pltpu.VMEM_SHARED((_NP, _D), jnp.float32),
      pltpu.VMEM((2, _CHUNK), jnp.int32),
      pltpu.VMEM((_CHUNK, _D), jnp.float32),
      pltpu.VMEM((_CHUNK, _D), jnp.float32),
      pltpu.SemaphoreType.DMA,
      pltpu.SemaphoreType.DMA,
      pltpu.SemaphoreType.DMA,
      pltpu.SemaphoreType.DMA,
  ]
  if with_gather:
    scratch.append(pltpu.VMEM((2, _CHUNK), jnp.int32))

  def body(*refs):
    if with_gather:
      (table, srcs, dsts, z128, o_acc,
       acc_s, dst_v, rows_a, rows_b, gsa, gsb, ssa, ssb, src_v) = refs
    else:
      (ones, dsts, z128, o_acc,
       acc_s, dst_v, rows_a, rows_b, gsa, gsb, ssa, ssb) = refs
    c = lax.axis_index("c")
    s = lax.axis_index("s")
    wid = s * _NC + c
    r0 = s * _RPT

    pltpu.sync_copy(z128, rows_a)
    for k in range(_RPT // _CHUNK):
      pltpu.sync_copy(rows_a, acc_s.at[pl.ds(r0 + k * _CHUNK, _CHUNK)])
    if not with_gather:
      pltpu.sync_copy(ones, rows_a)
      pltpu.sync_copy(ones, rows_b)
    plsc.subcore_barrier()

    e0 = wid * _EPW

    if with_gather:
      def step2(jj, carry):
        b0 = e0 + jj * (2 * _CHUNK)
        b1 = b0 + _CHUNK
        pltpu.sync_copy(dsts.at[pl.ds(b0, _CHUNK)], dst_v.at[0])
        pltpu.sync_copy(srcs.at[pl.ds(b0, _CHUNK)], src_v.at[0])
        ga = pltpu.async_copy(table.at[src_v.at[0]], rows_a, gsa)
        pltpu.sync_copy(dsts.at[pl.ds(b1, _CHUNK)], dst_v.at[1])
        pltpu.sync_copy(srcs.at[pl.ds(b1, _CHUNK)], src_v.at[1])
        gb = pltpu.async_copy(table.at[src_v.at[1]], rows_b, gsb)
        ga.wait()
        sa = pltpu.async_copy(rows_a, acc_s.at[dst_v.at[0]], ssa, add=True)
        gb.wait()
        sb = pltpu.async_copy(rows_b, acc_s.at[dst_v.at[1]], ssb, add=True)
        sa.wait()
        sb.wait()
        return carry
      lax.fori_loop(0, _STEPS // 2, step2, 0)
      base = e0 + (_STEPS - 1) * _CHUNK
      pltpu.sync_copy(dsts.at[pl.ds(base, _CHUNK)], dst_v.at[0])
      pltpu.sync_copy(srcs.at[pl.ds(base, _CHUNK)], src_v.at[0])
      pltpu.async_copy(table.at[src_v.at[0]], rows_a, gsa).wait()
      pltpu.sync_copy(rows_a, acc_s.at[dst_v.at[0]], add=True)
    else:
      def step2(jj, carry):
        b0 = e0 + jj * (2 * _CHUNK)
        b1 = b0 + _CHUNK
        pltpu.sync_copy(dsts.at[pl.ds(b0, _CHUNK)], dst_v.at[0])
        pltpu.sync_copy(dsts.at[pl.ds(b1, _CHUNK)], dst_v.at[1])
        sa = pltpu.async_copy(rows_a, acc_s.at[dst_v.at[0]], ssa, add=True)
        sb = pltpu.async_copy(rows_b, acc_s.at[dst_v.at[1]], ssb, add=True)
        sa.wait()
        sb.wait()
        return carry
      lax.fori_loop(0, _STEPS // 2, step2, 0)
      base = e0 + (_STEPS - 1) * _CHUNK
      pltpu.sync_copy(dsts.at[pl.ds(base, _CHUNK)], dst_v.at[0])
      pltpu.sync_copy(rows_a, acc_s.at[dst_v.at[0]], add=True)
    plsc.subcore_barrier()

    for k in range(_RPT // _CHUNK):
      pltpu.sync_copy(acc_s.at[pl.ds(r0 + k * _CHUNK, _CHUNK)], rows_a)
      pltpu.sync_copy(rows_a, o_acc.at[pl.ds(c * _NP + r0 + k * _CHUNK, _CHUNK)])

  return pl.kernel(body, out_type=out_type, mesh=mesh, scratch_types=scratch)


_agg = _make_agg(True)
_deg_pass = _make_agg(False)

_RB = 2000


def _conv_body(acc_ref, deg_ref, x_ref, wl_ref, wr_ref, b_ref, o_ref, *, relu):
  ssum = acc_ref[0] + acc_ref[1]
  deg = deg_ref[0, :, 0:1] + deg_ref[1, :, 0:1]
  mean = ssum / jnp.maximum(deg, 1.0)
  out = (jnp.dot(mean, wl_ref[...], preferred_element_type=jnp.float32)
         + jnp.dot(x_ref[...], wr_ref[...], preferred_element_type=jnp.float32)
         + b_ref[...])
  nrm = jnp.sqrt(jnp.sum(out * out, axis=-1, keepdims=True))
  out = out / jnp.maximum(nrm, 1e-12)
  if relu:
    out = jnp.maximum(out, 0.0)
  o_ref[...] = out


def _final_body(acc_ref, deg_ref, x_ref, wl_ref, wr_ref, b_ref, wfc_ref,
                bfc_ref, o_ref):
  ssum = acc_ref[0] + acc_ref[1]
  deg = deg_ref[0, :, 0:1] + deg_ref[1, :, 0:1]
  mean = ssum / jnp.maximum(deg, 1.0)
  out = (jnp.dot(mean, wl_ref[...], preferred_element_type=jnp.float32)
         + jnp.dot(x_ref[...], wr_ref[...], preferred_element_type=jnp.float32)
         + b_ref[...])
  nrm = jnp.sqrt(jnp.sum(out * out, axis=-1, keepdims=True))
  out = out / jnp.maximum(nrm, 1e-12)
  logits = (jnp.dot(out, wfc_ref[...], preferred_element_type=jnp.float32)
            + bfc_ref[...])
  m = jnp.max(logits, axis=-1, keepdims=True)
  e = jnp.exp(logits - m)
  o_ref[...] = e / jnp.sum(e, axis=-1, keepdims=True)


def _conv(acc, deg, x, wl, wr, b, relu):
  grid = (_N // _RB,)
  return pl.pallas_call(
      functools.partial(_conv_body, relu=relu),
      grid=grid,
      in_specs=[
          pl.BlockSpec((_NC, _RB, _D), lambda i: (0, i, 0)),
          pl.BlockSpec((_NC, _RB, _D), lambda i: (0, i, 0)),
          pl.BlockSpec((_RB, _D), lambda i: (i, 0)),
          pl.BlockSpec((_D, _D), lambda i: (0, 0)),
          pl.BlockSpec((_D, _D), lambda i: (0, 0)),
          pl.BlockSpec((1, _D), lambda i: (0, 0)),
      ],
      out_specs=pl.BlockSpec((_RB, _D), lambda i: (i, 0)),
      out_shape=jax.ShapeDtypeStruct((_N, _D), jnp.float32),
  )(acc, deg, x, wl, wr, b)


def _final(acc, deg, x, wl, wr, b, wfc, bfc):
  grid = (_N // _RB,)
  return pl.pallas_call(
      _final_body,
      grid=grid,
      in_specs=[
          pl.BlockSpec((_NC, _RB, _D), lambda i: (0, i, 0)),
          pl.BlockSpec((_NC, _RB, _D), lambda i: (0, i, 0)),
          pl.BlockSpec((_RB, _D), lambda i: (i, 0)),
          pl.BlockSpec((_D, _D), lambda i: (0, 0)),
          pl.BlockSpec((_D, _D), lambda i: (0, 0)),
          pl.BlockSpec((1, _D), lambda i: (0, 0)),
          pl.BlockSpec((_D, _DOUT), lambda i: (0, 0)),
          pl.BlockSpec((1, _DOUT), lambda i: (0, 0)),
      ],
      out_specs=pl.BlockSpec((_RB, _DOUT), lambda i: (i, 0)),
      out_shape=jax.ShapeDtypeStruct((_N, _DOUT), jnp.float32),
  )(acc, deg, x, wl, wr, b, wfc, bfc)


def kernel(x, edge_index, W_l1, W_r1, b1, W_l2, W_r2, b2, W_l3, W_r3, b3,
           W_fc, b_fc):
  src = edge_index[0].astype(jnp.int32)
  dst = edge_index[1].astype(jnp.int32)
  z128 = jnp.zeros((_CHUNK, _D), jnp.float32)
  ones = jnp.ones((_CHUNK, _D), jnp.float32)

  deg = _deg_pass(ones, dst, z128).reshape(_NC, _NP, _D)
  acc1 = _agg(x, src, dst, z128).reshape(_NC, _NP, _D)
  h1 = _conv(acc1, deg, x, W_l1, W_r1, b1.reshape(1, _D), relu=True)
  acc2 = _agg(h1, src, dst, z128).reshape(_NC, _NP, _D)
  h2 = _conv(acc2, deg, h1, W_l2, W_r2, b2.reshape(1, _D), relu=True)
  acc3 = _agg(h2, src, dst, z128).reshape(_NC, _NP, _D)
  return _final(acc3, deg, h2, W_l3, W_r3, b3.reshape(1, _D),
                W_fc, b_fc.reshape(1, _DOUT))


# speedup vs baseline: 7.7485x; 1.2433x over previous
import functools

import jax
import jax.numpy as jnp
from jax import lax
from jax.experimental import pallas as pl
from jax.experimental.pallas import tpu as pltpu
from jax.experimental.pallas import tpu_sc as plsc

_N = 10000
_D = 128
_E = 320000
_DOUT = 64

_NC = 2
_NS = 16
_NW = _NC * _NS
_EPW = _E // _NW
_CHUNK = 128
_FULL = _EPW // _CHUNK
_TAIL = _EPW - _FULL * _CHUNK
_NP = 10240
_RPT = _NP // _NS


def _make_agg(with_gather):
  mesh = plsc.VectorSubcoreMesh(core_axis_name="c", subcore_axis_name="s")
  out_type = jax.ShapeDtypeStruct((_NC * _NP, _D), jnp.float32)
  scratch = [
      pltpu.VMEM_SHARED((_NP, _D), jnp.float32),
      pltpu.VMEM((2, _CHUNK), jnp.int32),
      pltpu.VMEM((_CHUNK, _D), jnp.float32),
      pltpu.VMEM((_CHUNK, _D), jnp.float32),
      pltpu.SemaphoreType.DMA,
      pltpu.SemaphoreType.DMA,
      pltpu.SemaphoreType.DMA,
      pltpu.SemaphoreType.DMA,
      pltpu.VMEM((_TAIL,), jnp.int32),
      pltpu.VMEM((_TAIL, _D), jnp.float32),
  ]
  if with_gather:
    scratch.append(pltpu.VMEM((2, _CHUNK), jnp.int32))
    scratch.append(pltpu.VMEM((_TAIL,), jnp.int32))

  def body(*refs):
    if with_gather:
      (table, srcs, dsts, z128, o_acc,
       acc_s, dst_v, rows_a, rows_b, gsa, gsb, ssa, ssb,
       dst_t, rows_t, src_v, src_t) = refs
    else:
      (ones, dsts, z128, o_acc,
       acc_s, dst_v, rows_a, rows_b, gsa, gsb, ssa, ssb,
       dst_t, rows_t) = refs
    c = lax.axis_index("c")
    s = lax.axis_index("s")
    wid = s * _NC + c
    r0 = s * _RPT
    nzd = _RPT // _CHUNK

    pltpu.sync_copy(z128, rows_a)
    zs = [pltpu.async_copy(rows_a, acc_s.at[pl.ds(r0 + k * _CHUNK, _CHUNK)],
                           ssa) for k in range(nzd)]
    for h in zs:
      h.wait()
    if not with_gather:
      pltpu.sync_copy(ones, rows_a)
      pltpu.sync_copy(ones, rows_b)
      pltpu.sync_copy(ones.at[pl.ds(0, _TAIL)], rows_t)
    plsc.subcore_barrier()

    e0 = wid * _EPW

    if with_gather:
      def step2(jj, carry):
        b0 = e0 + jj * (2 * _CHUNK)
        b1 = b0 + _CHUNK
        pltpu.sync_copy(dsts.at[pl.ds(b0, _CHUNK)], dst_v.at[0])
        pltpu.sync_copy(srcs.at[pl.ds(b0, _CHUNK)], src_v.at[0])
        ga = pltpu.async_copy(table.at[src_v.at[0]], rows_a, gsa)
        pltpu.sync_copy(dsts.at[pl.ds(b1, _CHUNK)], dst_v.at[1])
        pltpu.sync_copy(srcs.at[pl.ds(b1, _CHUNK)], src_v.at[1])
        gb = pltpu.async_copy(table.at[src_v.at[1]], rows_b, gsb)
        ga.wait()
        sa = pltpu.async_copy(rows_a, acc_s.at[dst_v.at[0]], ssa, add=True)
        gb.wait()
        sb = pltpu.async_copy(rows_b, acc_s.at[dst_v.at[1]], ssb, add=True)
        sa.wait()
        sb.wait()
        return carry
      lax.fori_loop(0, _FULL // 2, step2, 0)
      base = e0 + _FULL * _CHUNK
      pltpu.sync_copy(dsts.at[pl.ds(base, _TAIL)], dst_t)
      pltpu.sync_copy(srcs.at[pl.ds(base, _TAIL)], src_t)
      pltpu.async_copy(table.at[src_t], rows_t, gsa).wait()
      pltpu.sync_copy(rows_t, acc_s.at[dst_t], add=True)
    else:
      def step2(jj, carry):
        b0 = e0 + jj * (2 * _CHUNK)
        b1 = b0 + _CHUNK
        pltpu.sync_copy(dsts.at[pl.ds(b0, _CHUNK)], dst_v.at[0])
        pltpu.sync_copy(dsts.at[pl.ds(b1, _CHUNK)], dst_v.at[1])
        sa = pltpu.async_copy(rows_a, acc_s.at[dst_v.at[0]], ssa, add=True)
        sb = pltpu.async_copy(rows_b, acc_s.at[dst_v.at[1]], ssb, add=True)
        sa.wait()
        sb.wait()
        return carry
      lax.fori_loop(0, _FULL // 2, step2, 0)
      base = e0 + _FULL * _CHUNK
      pltpu.sync_copy(dsts.at[pl.ds(base, _TAIL)], dst_t)
      pltpu.sync_copy(rows_t, acc_s.at[dst_t], add=True)
    plsc.subcore_barrier()

    bufs = (rows_a, rows_b)
    gsems = (gsa, gsb)
    osems = (ssa, ssb)
    ins = [None, None]
    outs = [None, None]
    ins[0] = pltpu.async_copy(acc_s.at[pl.ds(r0, _CHUNK)], bufs[0], gsems[0])
    for k in range(nzd):
      b = k % 2
      if k + 1 < nzd:
        o = 1 - b
        if outs[o] is not None:
          outs[o].wait()
          outs[o] = None
        ins[o] = pltpu.async_copy(
            acc_s.at[pl.ds(r0 + (k + 1) * _CHUNK, _CHUNK)], bufs[o], gsems[o])
      ins[b].wait()
      outs[b] = pltpu.async_copy(
          bufs[b], o_acc.at[pl.ds(c * _NP + r0 + k * _CHUNK, _CHUNK)],
          osems[b])
    for o in outs:
      if o is not None:
        o.wait()

  return pl.kernel(body, out_type=out_type, mesh=mesh, scratch_types=scratch)


_agg = _make_agg(True)
_deg_pass = _make_agg(False)

_RB = 2000


def _conv_body(acc_ref, deg_ref, x_ref, wl_ref, wr_ref, b_ref, o_ref, *, relu):
  ssum = acc_ref[0] + acc_ref[1]
  deg = deg_ref[0, :, 0:1] + deg_ref[1, :, 0:1]
  mean = ssum / jnp.maximum(deg, 1.0)
  out = (jnp.dot(mean, wl_ref[...], preferred_element_type=jnp.float32)
         + jnp.dot(x_ref[...], wr_ref[...], preferred_element_type=jnp.float32)
         + b_ref[...])
  nrm = jnp.sqrt(jnp.sum(out * out, axis=-1, keepdims=True))
  out = out / jnp.maximum(nrm, 1e-12)
  if relu:
    out = jnp.maximum(out, 0.0)
  o_ref[...] = out


def _final_body(acc_ref, deg_ref, x_ref, wl_ref, wr_ref, b_ref, wfc_ref,
                bfc_ref, o_ref):
  ssum = acc_ref[0] + acc_ref[1]
  deg = deg_ref[0, :, 0:1] + deg_ref[1, :, 0:1]
  mean = ssum / jnp.maximum(deg, 1.0)
  out = (jnp.dot(mean, wl_ref[...], preferred_element_type=jnp.float32)
         + jnp.dot(x_ref[...], wr_ref[...], preferred_element_type=jnp.float32)
         + b_ref[...])
  nrm = jnp.sqrt(jnp.sum(out * out, axis=-1, keepdims=True))
  out = out / jnp.maximum(nrm, 1e-12)
  logits = (jnp.dot(out, wfc_ref[...], preferred_element_type=jnp.float32)
            + bfc_ref[...])
  m = jnp.max(logits, axis=-1, keepdims=True)
  e = jnp.exp(logits - m)
  o_ref[...] = e / jnp.sum(e, axis=-1, keepdims=True)


def _conv(acc, deg, x, wl, wr, b, relu):
  grid = (_N // _RB,)
  return pl.pallas_call(
      functools.partial(_conv_body, relu=relu),
      grid=grid,
      in_specs=[
          pl.BlockSpec((_NC, _RB, _D), lambda i: (0, i, 0)),
          pl.BlockSpec((_NC, _RB, _D), lambda i: (0, i, 0)),
          pl.BlockSpec((_RB, _D), lambda i: (i, 0)),
          pl.BlockSpec((_D, _D), lambda i: (0, 0)),
          pl.BlockSpec((_D, _D), lambda i: (0, 0)),
          pl.BlockSpec((1, _D), lambda i: (0, 0)),
      ],
      out_specs=pl.BlockSpec((_RB, _D), lambda i: (i, 0)),
      out_shape=jax.ShapeDtypeStruct((_N, _D), jnp.float32),
  )(acc, deg, x, wl, wr, b)


def _final(acc, deg, x, wl, wr, b, wfc, bfc):
  grid = (_N // _RB,)
  return pl.pallas_call(
      _final_body,
      grid=grid,
      in_specs=[
          pl.BlockSpec((_NC, _RB, _D), lambda i: (0, i, 0)),
          pl.BlockSpec((_NC, _RB, _D), lambda i: (0, i, 0)),
          pl.BlockSpec((_RB, _D), lambda i: (i, 0)),
          pl.BlockSpec((_D, _D), lambda i: (0, 0)),
          pl.BlockSpec((_D, _D), lambda i: (0, 0)),
          pl.BlockSpec((1, _D), lambda i: (0, 0)),
          pl.BlockSpec((_D, _DOUT), lambda i: (0, 0)),
          pl.BlockSpec((1, _DOUT), lambda i: (0, 0)),
      ],
      out_specs=pl.BlockSpec((_RB, _DOUT), lambda i: (i, 0)),
      out_shape=jax.ShapeDtypeStruct((_N, _DOUT), jnp.float32),
  )(acc, deg, x, wl, wr, b, wfc, bfc)


def kernel(x, edge_index, W_l1, W_r1, b1, W_l2, W_r2, b2, W_l3, W_r3, b3,
           W_fc, b_fc):
  src = edge_index[0].astype(jnp.int32)
  dst = edge_index[1].astype(jnp.int32)
  z128 = jnp.zeros((_CHUNK, _D), jnp.float32)
  ones = jnp.ones((_CHUNK, _D), jnp.float32)

  deg = _deg_pass(ones, dst, z128).reshape(_NC, _NP, _D)
  acc1 = _agg(x, src, dst, z128).reshape(_NC, _NP, _D)
  h1 = _conv(acc1, deg, x, W_l1, W_r1, b1.reshape(1, _D), relu=True)
  acc2 = _agg(h1, src, dst, z128).reshape(_NC, _NP, _D)
  h2 = _conv(acc2, deg, h1, W_l2, W_r2, b2.reshape(1, _D), relu=True)
  acc3 = _agg(h2, src, dst, z128).reshape(_NC, _NP, _D)
  return _final(acc3, deg, h2, W_l3, W_r3, b3.reshape(1, _D),
                W_fc, b_fc.reshape(1, _DOUT))
